# Initial kernel scaffold; baseline (speedup 1.0000x reference)
import jax
import jax.numpy as jnp
from jax.experimental import pallas as pl


def kernel(x, enc_W1, enc_b1, enc_W2, enc_b2, enc_W3, enc_b3, enc_W4, enc_b4, pool_lin_W, pool_lin_b, pool_att_W, pool_att_b, score_W1, score_b1, score_W2, score_W3, gnn_W1, gnn_b1, gnn_W2, gnn_b2, gnn_W3, gnn_b3, gnn_W4, gnn_b4, cls_W, cls_b, edge_index, batch):
    raise NotImplementedError("write your pallas kernel here")


# speedup vs baseline: 1.1280x; 1.1280x over previous
import functools
import math

import jax
import jax.numpy as jnp
from jax.experimental import pallas as pl
from jax.experimental.pallas import tpu as pltpu

N = 10000
D = 128
NG = 128
NC = 10
KSEL = N // 2

BLK = 128
NPAD = 10112


def _gin_mlp_body(scale, h_ref, a0_ref, a1_ref, w1_ref, b1_ref, w2_ref, b2_ref, o_ref):
    z = h_ref[...] + scale * (a0_ref[...] + a1_ref[...])
    z = jnp.maximum(jnp.dot(z, w1_ref[...], preferred_element_type=jnp.float32) + b1_ref[...], 0.0)
    o_ref[...] = jnp.maximum(jnp.dot(z, w2_ref[...], preferred_element_type=jnp.float32) + b2_ref[...], 0.0)


def _gin_mlp(h, a0, a1, w1, b1, w2, b2, scale=1.0):
    grid = (NPAD // BLK,)
    row_spec = pl.BlockSpec((BLK, D), lambda i: (i, 0))
    w_spec = pl.BlockSpec((D, D), lambda i: (0, 0))
    b_spec = pl.BlockSpec((1, D), lambda i: (0, 0))
    return pl.pallas_call(
        functools.partial(_gin_mlp_body, scale),
        grid=grid,
        in_specs=[row_spec, row_spec, row_spec, w_spec, b_spec, w_spec, b_spec],
        out_specs=row_spec,
        out_shape=jax.ShapeDtypeStruct((NPAD, D), jnp.float32),
    )(h, a0, a1, w1, b1.reshape(1, D), w2, b2.reshape(1, D))


def _pad_rows(x):
    return jnp.pad(x, ((0, NPAD - N), (0, 0)))


def kernel(x, enc_W1, enc_b1, enc_W2, enc_b2, enc_W3, enc_b3, enc_W4, enc_b4,
           pool_lin_W, pool_lin_b, pool_att_W, pool_att_b,
           score_W1, score_b1, score_W2, score_W3,
           gnn_W1, gnn_b1, gnn_W2, gnn_b2, gnn_W3, gnn_b3, gnn_W4, gnn_b4,
           cls_W, cls_b, edge_index, batch):
    src, dst = edge_index[0], edge_index[1]
    zero = jnp.zeros((NPAD, D), jnp.float32)

    def gin(h, w1, b1, w2, b2, scale=1.0, ew=None, s=src, d=dst):
        msg = h[s]
        if ew is not None:
            msg = msg * ew[:, None]
        agg = jax.ops.segment_sum(msg, d, num_segments=N)
        return _gin_mlp(_pad_rows(h), _pad_rows(agg), zero, w1, b1, w2, b2, scale)[:N]

    h = gin(x, enc_W1, enc_b1, enc_W2, enc_b2)
    h = gin(h, enc_W3, enc_b3, enc_W4, enc_b4)

    loop = jnp.arange(N, dtype=src.dtype)
    s2 = jnp.concatenate([src, loop])
    d2 = jnp.concatenate([dst, loop])
    xpj = h[s2]
    xq = jax.ops.segment_max(xpj, d2, num_segments=N)
    xq = (xq @ pool_lin_W + pool_lin_b)[d2]
    sc = (jnp.concatenate([xq, xpj], axis=-1) @ pool_att_W + pool_att_b)[:, 0]
    sc = jnp.where(sc > 0, sc, 0.2 * sc)
    m = jax.ops.segment_max(sc, d2, num_segments=N)
    e = jnp.exp(sc - m[d2])
    den = jax.ops.segment_sum(e, d2, num_segments=N)
    attn = e / (den[d2] + 1e-16)
    xc = jax.ops.segment_sum(h[s2] * attn[:, None], d2, num_segments=N)
    t2 = (xc @ score_W2)[d2]
    t3 = (xc @ score_W3)[s2]
    fit = (xc @ score_W1 + score_b1)[:, 0] + jax.ops.segment_sum((t2 - t3)[:, 0], d2, num_segments=N)
    fitness = jax.nn.sigmoid(fit)
    topv, perm = jax.lax.top_k(fitness, KSEL)
    kept = jnp.zeros((N,), jnp.float32).at[perm].set(1.0)
    w = kept * fitness
    px = xc * w[:, None]
    em = kept[src] * kept[dst]
    c = float(1.0 / (1.0 + math.exp(-1.0)))

    g = gin(px, gnn_W1, gnn_b1, gnn_W2, gnn_b2, scale=c, ew=em)
    g = gin(g, gnn_W3, gnn_b3, gnn_W4, gnn_b4, scale=c, ew=em)

    sums = jax.ops.segment_sum(g * kept[:, None], batch, num_segments=NG)
    cnt = jax.ops.segment_sum(kept, batch, num_segments=NG)
    readout = sums / jnp.maximum(cnt, 1.0)[:, None]
    return readout @ cls_W + cls_b


# speedup vs baseline: 1.3665x; 1.2115x over previous
import functools
import math

import jax
import jax.numpy as jnp
from jax import lax
from jax.experimental import pallas as pl
from jax.experimental.pallas import tpu as pltpu
from jax.experimental.pallas import tpu_sc as plsc

N = 10000
D = 128
NG = 128
NC = 10
KSEL = N // 2

BLK = 128
NPAD = 10112

SC_CORES = 2
SC_TILES = 16
NWORK = SC_CORES * SC_TILES
EPAD = 327680
EPW = EPAD // NWORK
UNITS = EPW // 128
ROWS_PT = NPAD // SC_TILES


def _segsum_sc_body(h_hbm, srcp, dstp, zin, out, acc, sidx, didx, rows, sem):
    cid = lax.axis_index("c")
    sid = lax.axis_index("s")
    wid = sid * SC_CORES + cid
    r0 = sid * ROWS_PT
    pltpu.sync_copy(zin.at[pl.ds(r0, ROWS_PT)], acc.at[pl.ds(r0, ROWS_PT)])
    plsc.subcore_barrier()
    wbase = wid * EPW

    def body(u, carry):
        base = wbase + u * 128
        pltpu.sync_copy(srcp.at[pl.ds(base, 128)], sidx)
        pltpu.sync_copy(dstp.at[pl.ds(base, 128)], didx)
        pltpu.async_copy(h_hbm.at[sidx], rows, sem).wait()
        pltpu.sync_copy(rows, acc.at[didx], add=True)
        return carry

    lax.fori_loop(0, UNITS, body, 0)
    plsc.subcore_barrier()
    pltpu.sync_copy(acc.at[pl.ds(r0, ROWS_PT)], out.at[cid, pl.ds(r0, ROWS_PT)])


def _segsum_sc(hp, srcp, dstp):
    zin = jnp.zeros((NPAD, D), jnp.float32)
    mesh = plsc.VectorSubcoreMesh(core_axis_name="c", subcore_axis_name="s",
                                  num_cores=SC_CORES, num_subcores=SC_TILES)
    return pl.kernel(
        _segsum_sc_body,
        out_type=jax.ShapeDtypeStruct((SC_CORES, NPAD, D), jnp.float32),
        mesh=mesh,
        scratch_types=[
            pltpu.VMEM_SHARED((NPAD, D), jnp.float32),
            pltpu.VMEM((128,), jnp.int32),
            pltpu.VMEM((128,), jnp.int32),
            pltpu.VMEM((128, D), jnp.float32),
            pltpu.SemaphoreType.DMA,
        ],
    )(hp, srcp, dstp, zin)


def _gin_mlp_body(scale, h_ref, a0_ref, a1_ref, w1_ref, b1_ref, w2_ref, b2_ref, o_ref):
    z = h_ref[...] + scale * (a0_ref[...] + a1_ref[...])
    z = jnp.maximum(jnp.dot(z, w1_ref[...], preferred_element_type=jnp.float32) + b1_ref[...], 0.0)
    o_ref[...] = jnp.maximum(jnp.dot(z, w2_ref[...], preferred_element_type=jnp.float32) + b2_ref[...], 0.0)


def _gin_mlp(h, a0, a1, w1, b1, w2, b2, scale=1.0):
    grid = (NPAD // BLK,)
    row_spec = pl.BlockSpec((BLK, D), lambda i: (i, 0))
    w_spec = pl.BlockSpec((D, D), lambda i: (0, 0))
    b_spec = pl.BlockSpec((1, D), lambda i: (0, 0))
    return pl.pallas_call(
        functools.partial(_gin_mlp_body, scale),
        grid=grid,
        in_specs=[row_spec, row_spec, row_spec, w_spec, b_spec, w_spec, b_spec],
        out_specs=row_spec,
        out_shape=jax.ShapeDtypeStruct((NPAD, D), jnp.float32),
    )(h, a0, a1, w1, b1.reshape(1, D), w2, b2.reshape(1, D))


def _pad_rows(x):
    return jnp.pad(x, ((0, NPAD - N), (0, 0)))


def kernel(x, enc_W1, enc_b1, enc_W2, enc_b2, enc_W3, enc_b3, enc_W4, enc_b4,
           pool_lin_W, pool_lin_b, pool_att_W, pool_att_b,
           score_W1, score_b1, score_W2, score_W3,
           gnn_W1, gnn_b1, gnn_W2, gnn_b2, gnn_W3, gnn_b3, gnn_W4, gnn_b4,
           cls_W, cls_b, edge_index, batch):
    src, dst = edge_index[0], edge_index[1]
    epad = jnp.full((EPAD - src.shape[0],), N, jnp.int32)
    srcp = jnp.concatenate([src, epad])
    dstp = jnp.concatenate([dst, epad])

    def gin(hp, dp, w1, b1, w2, b2, scale=1.0):
        p = _segsum_sc(hp, srcp, dp)
        return _gin_mlp(hp, p[0], p[1], w1, b1, w2, b2, scale)

    hp = gin(_pad_rows(x), dstp, enc_W1, enc_b1, enc_W2, enc_b2)
    hp = gin(hp, dstp, enc_W3, enc_b3, enc_W4, enc_b4)
    h = hp[:N]

    loop = jnp.arange(N, dtype=src.dtype)
    s2 = jnp.concatenate([src, loop])
    d2 = jnp.concatenate([dst, loop])
    xpj = h[s2]
    xq = jax.ops.segment_max(xpj, d2, num_segments=N)
    xq = (xq @ pool_lin_W + pool_lin_b)[d2]
    sc = (jnp.concatenate([xq, xpj], axis=-1) @ pool_att_W + pool_att_b)[:, 0]
    sc = jnp.where(sc > 0, sc, 0.2 * sc)
    m = jax.ops.segment_max(sc, d2, num_segments=N)
    e = jnp.exp(sc - m[d2])
    den = jax.ops.segment_sum(e, d2, num_segments=N)
    attn = e / (den[d2] + 1e-16)
    xc = jax.ops.segment_sum(h[s2] * attn[:, None], d2, num_segments=N)
    t2 = (xc @ score_W2)[d2]
    t3 = (xc @ score_W3)[s2]
    fit = (xc @ score_W1 + score_b1)[:, 0] + jax.ops.segment_sum((t2 - t3)[:, 0], d2, num_segments=N)
    fitness = jax.nn.sigmoid(fit)
    topv, perm = jax.lax.top_k(fitness, KSEL)
    kept = jnp.zeros((N,), jnp.float32).at[perm].set(1.0)
    w = kept * fitness
    px = xc * w[:, None]
    em = kept[src] * kept[dst]
    c = float(1.0 / (1.0 + math.exp(-1.0)))
    dstm = jnp.concatenate([jnp.where(em > 0.5, dst, N).astype(jnp.int32), epad])

    gp = gin(_pad_rows(px), dstm, gnn_W1, gnn_b1, gnn_W2, gnn_b2, scale=c)
    gp = gin(gp, dstm, gnn_W3, gnn_b3, gnn_W4, gnn_b4, scale=c)
    g = gp[:N]

    sums = jax.ops.segment_sum(g * kept[:, None], batch, num_segments=NG)
    cnt = jax.ops.segment_sum(kept, batch, num_segments=NG)
    readout = sums / jnp.maximum(cnt, 1.0)[:, None]
    return readout @ cls_W + cls_b


# speedup vs baseline: 2.4274x; 1.7763x over previous
import functools
import math

import jax
import jax.numpy as jnp
from jax import lax
from jax.experimental import pallas as pl
from jax.experimental.pallas import tpu as pltpu
from jax.experimental.pallas import tpu_sc as plsc

N = 10000
D = 128
NG = 128
NC = 10
KSEL = N // 2

BLK = 128
NPAD = 10240

SC_CORES = 2
SC_TILES = 16
NWORK = SC_CORES * SC_TILES
EPAD = 327680
EPW = EPAD // NWORK
UNITS = EPW // 128
ROWS_PT = NPAD // SC_TILES
E = 320000
RNG = NPAD // NWORK
SCAP = 16352
CH = 2048
NCH = 157


def _segsum_sc_body(h_hbm, srcp, dstp, zin, out, acc, sidx, didx, rows, sem):
    cid = lax.axis_index("c")
    sid = lax.axis_index("s")
    wid = sid * SC_CORES + cid
    r0 = sid * ROWS_PT
    pltpu.sync_copy(zin.at[pl.ds(r0, ROWS_PT)], acc.at[pl.ds(r0, ROWS_PT)])
    plsc.subcore_barrier()
    wbase = wid * EPW

    def body(u, carry):
        base = wbase + u * 128
        pltpu.sync_copy(srcp.at[pl.ds(base, 128)], sidx)
        pltpu.sync_copy(dstp.at[pl.ds(base, 128)], didx)
        pltpu.async_copy(h_hbm.at[sidx], rows, sem).wait()
        pltpu.sync_copy(rows, acc.at[didx], add=True)
        return carry

    lax.fori_loop(0, UNITS, body, 0)
    plsc.subcore_barrier()
    pltpu.sync_copy(acc.at[pl.ds(r0, ROWS_PT)], out.at[cid, pl.ds(r0, ROWS_PT)])


def _segsum_sc(hp, srcp, dstp):
    zin = jnp.zeros((NPAD, D), jnp.float32)
    mesh = plsc.VectorSubcoreMesh(core_axis_name="c", subcore_axis_name="s",
                                  num_cores=SC_CORES, num_subcores=SC_TILES)
    return pl.kernel(
        _segsum_sc_body,
        out_type=jax.ShapeDtypeStruct((SC_CORES, NPAD, D), jnp.float32),
        mesh=mesh,
        scratch_types=[
            pltpu.VMEM_SHARED((NPAD, D), jnp.float32),
            pltpu.VMEM((128,), jnp.int32),
            pltpu.VMEM((128,), jnp.int32),
            pltpu.VMEM((128, D), jnp.float32),
            pltpu.SemaphoreType.DMA,
        ],
    )(hp, srcp, dstp, zin)


def _leaky(t):
    return jnp.where(t > 0, t, 0.2 * t)


def _sload(ref, i):
    return ref[pl.ds(i, 16)][0]


def _sstore(ref, i, val):
    lane0 = lax.iota(jnp.int32, 16) == 0
    plsc.store_scatter(ref, [jnp.full((16,), i, jnp.int32)],
                       jnp.full((16,), val, ref.dtype), mask=lane0)


def _poolmax_body(h_hbm, src_h, dst_h, v_hbm, wa_hbm, u_out, m_out,
                  acc, ss, sd, svm, dvm, v_vm, wa_vm, u_vm, m_vm, rows, sem):
    cid = lax.axis_index("c")
    sid = lax.axis_index("s")
    wid = sid * SC_CORES + cid
    lo = wid * RNG
    pltpu.sync_copy(h_hbm.at[pl.ds(lo, RNG)], acc.at[pl.ds(0, RNG)])
    pltpu.sync_copy(v_hbm, v_vm.at[pl.ds(0, NPAD)])
    pltpu.sync_copy(wa_hbm, wa_vm)
    dummy_s = jnp.full((16,), N, jnp.int32)
    dummy_d = jnp.full((16,), RNG, jnp.int32)

    def prefill(j, carry):
        ss[pl.ds(j * 16, 16)] = dummy_s
        sd[pl.ds(j * 16, 16)] = dummy_d
        return carry

    lax.fori_loop(0, (SCAP + 160) // 16, prefill, 0)

    tail = E - (NCH - 1) * CH

    def chunk_body(ch, pos):
        base = ch * CH
        pltpu.sync_copy(src_h.at[pl.ds(base, tail)], svm.at[pl.ds(0, tail)])
        pltpu.sync_copy(dst_h.at[pl.ds(base, tail)], dvm.at[pl.ds(0, tail)])

        @pl.when(ch < NCH - 1)
        def _():
            pltpu.sync_copy(src_h.at[pl.ds(base + tail, CH - tail)],
                            svm.at[pl.ds(tail, CH - tail)])
            pltpu.sync_copy(dst_h.at[pl.ds(base + tail, CH - tail)],
                            dvm.at[pl.ds(tail, CH - tail)])
        nsv = jnp.where(ch < NCH - 1, CH // 16, tail // 16)

        def sub_body(k, pos):
            s16 = svm[pl.ds(k * 16, 16)]
            d16 = dvm[pl.ds(k * 16, 16)]
            msk = (d16 >= lo) & (d16 < lo + RNG)
            cnt = jnp.sum(msk.astype(jnp.int32))
            p = jnp.minimum(pos, SCAP)
            plsc.store_compressed(ss.at[pl.ds(p, 16)], s16, mask=msk)
            plsc.store_compressed(sd.at[pl.ds(p, 16)], d16 - lo, mask=msk)
            return pos + cnt

        return lax.fori_loop(0, nsv, sub_body, pos)

    pos = lax.fori_loop(0, NCH, chunk_body, jnp.int32(0))
    pos = jnp.minimum(pos, SCAP)

    def flush(f, carry):
        pltpu.async_copy(h_hbm.at[ss.at[pl.ds(f * 128, 128)]], rows, sem).wait()

        def upd(j, carry):
            dl = _sload(sd, f * 128 + j)
            for cc in range(8):
                sl = pl.ds(cc * 16, 16)
                acc[dl, sl] = jnp.maximum(acc[dl, sl], rows[j, sl])
            return carry

        return lax.fori_loop(0, 128, upd, carry)

    lax.fori_loop(0, (pos + 127) // 128, flush, 0)

    def udot(r, carry):
        t = jnp.zeros((16,), jnp.float32)
        for cc in range(8):
            sl = pl.ds(cc * 16, 16)
            t = t + acc[r, sl] * wa_vm[sl]
        uu = jnp.sum(t)
        _sstore(u_vm, r, uu)
        _sstore(m_vm, r, _leaky(uu + _sload(v_vm, lo + r)))
        return carry

    lax.fori_loop(0, RNG, udot, 0)

    def mupd(j, carry):
        dl = _sload(sd, j)
        sc = _leaky(_sload(u_vm, dl) + _sload(v_vm, _sload(ss, j)))
        _sstore(m_vm, dl, jnp.maximum(_sload(m_vm, dl), sc))
        return carry

    lax.fori_loop(0, pos, mupd, 0)
    pltpu.sync_copy(u_vm.at[pl.ds(0, RNG)], u_out.at[pl.ds(lo, RNG)])
    pltpu.sync_copy(m_vm.at[pl.ds(0, RNG)], m_out.at[pl.ds(lo, RNG)])


def _poolmax_sc(hp, src_e, dst_e, v, wa):
    mesh = plsc.VectorSubcoreMesh(core_axis_name="c", subcore_axis_name="s",
                                  num_cores=SC_CORES, num_subcores=SC_TILES)
    return pl.kernel(
        _poolmax_body,
        out_type=(jax.ShapeDtypeStruct((NPAD,), jnp.float32),
                  jax.ShapeDtypeStruct((NPAD,), jnp.float32)),
        mesh=mesh,
        compiler_params=pltpu.CompilerParams(needs_layout_passes=False),
        scratch_types=[
            pltpu.VMEM((RNG + 8, D), jnp.float32),
            pltpu.VMEM((SCAP + 160,), jnp.int32),
            pltpu.VMEM((SCAP + 160,), jnp.int32),
            pltpu.VMEM((CH,), jnp.int32),
            pltpu.VMEM((CH,), jnp.int32),
            pltpu.VMEM((NPAD + 16,), jnp.float32),
            pltpu.VMEM((D,), jnp.float32),
            pltpu.VMEM((RNG + 16,), jnp.float32),
            pltpu.VMEM((RNG + 16,), jnp.float32),
            pltpu.VMEM((128, D), jnp.float32),
            pltpu.SemaphoreType.DMA,
        ],
    )(hp, src_e, dst_e, v, wa)


def _scalsum_body(src_h, dst_h, u_hbm, v_hbm, m_hbm, den_out, deg_out, e_out,
                  uvm, vvm, mvm, dacc, gacc, svm, dvm, ebuf):
    cid = lax.axis_index("c")
    sid = lax.axis_index("s")
    wid = sid * SC_CORES + cid
    pltpu.sync_copy(u_hbm, uvm)
    pltpu.sync_copy(v_hbm, vvm)
    pltpu.sync_copy(m_hbm, mvm)
    zz = jnp.zeros((16,), jnp.float32)

    def zinit(j, carry):
        dacc[pl.ds(j * 16, 16)] = zz
        gacc[pl.ds(j * 16, 16)] = zz
        return carry

    lax.fori_loop(0, NPAD // 16, zinit, 0)
    epw = E // NWORK
    ones = jnp.ones((16,), jnp.float32)

    def chunk_body(ch, carry):
        base = wid * epw + ch * CH
        csz = jnp.minimum(epw - ch * CH, CH)
        pltpu.sync_copy(src_h.at[pl.ds(base, 1808)], svm.at[pl.ds(0, 1808)])
        pltpu.sync_copy(dst_h.at[pl.ds(base, 1808)], dvm.at[pl.ds(0, 1808)])

        @pl.when(csz == CH)
        def _():
            pltpu.sync_copy(src_h.at[pl.ds(base + 1808, CH - 1808)],
                            svm.at[pl.ds(1808, CH - 1808)])
            pltpu.sync_copy(dst_h.at[pl.ds(base + 1808, CH - 1808)],
                            dvm.at[pl.ds(1808, CH - 1808)])

        def sub_body(k, carry):
            s16 = svm[pl.ds(k * 16, 16)]
            d16 = dvm[pl.ds(k * 16, 16)]
            ud = plsc.load_gather(uvm, [d16])
            vs = plsc.load_gather(vvm, [s16])
            md = plsc.load_gather(mvm, [d16])
            e = jnp.exp(_leaky(ud + vs) - md)
            ebuf[pl.ds(k * 16, 16)] = e
            plsc.addupdate_scatter(dacc, [d16], e)
            plsc.addupdate_scatter(gacc, [d16], ones)
            return carry

        lax.fori_loop(0, csz // 16, sub_body, carry)
        pltpu.sync_copy(ebuf.at[pl.ds(0, 1808)], e_out.at[pl.ds(base, 1808)])

        @pl.when(csz == CH)
        def _():
            pltpu.sync_copy(ebuf.at[pl.ds(1808, CH - 1808)],
                            e_out.at[pl.ds(base + 1808, CH - 1808)])
        return carry

    lax.fori_loop(0, (epw + CH - 1) // CH, chunk_body, 0)
    pltpu.sync_copy(dacc, den_out.at[wid])
    pltpu.sync_copy(gacc, deg_out.at[wid])


def _scalsum_sc(src_e, dst_e, u, v, m):
    mesh = plsc.VectorSubcoreMesh(core_axis_name="c", subcore_axis_name="s",
                                  num_cores=SC_CORES, num_subcores=SC_TILES)
    return pl.kernel(
        _scalsum_body,
        out_type=(jax.ShapeDtypeStruct((NWORK, NPAD), jnp.float32),
                  jax.ShapeDtypeStruct((NWORK, NPAD), jnp.float32),
                  jax.ShapeDtypeStruct((E,), jnp.float32)),
        mesh=mesh,
        compiler_params=pltpu.CompilerParams(needs_layout_passes=False),
        scratch_types=[
            pltpu.VMEM((NPAD,), jnp.float32),
            pltpu.VMEM((NPAD,), jnp.float32),
            pltpu.VMEM((NPAD,), jnp.float32),
            pltpu.VMEM((NPAD,), jnp.float32),
            pltpu.VMEM((NPAD,), jnp.float32),
            pltpu.VMEM((CH,), jnp.int32),
            pltpu.VMEM((CH,), jnp.int32),
            pltpu.VMEM((CH,), jnp.float32),
        ],
    )(src_e, dst_e, u, v, m)


def _wsegsum_body(h_hbm, srcp, dstp, e_hbm, r_hbm, zin, out,
                  acc, rvm, sidx, didx, ebuf, abuf, rows, sem):
    cid = lax.axis_index("c")
    sid = lax.axis_index("s")
    wid = sid * SC_CORES + cid
    r0 = sid * ROWS_PT
    pltpu.sync_copy(zin.at[pl.ds(r0, ROWS_PT)], acc.at[pl.ds(r0, ROWS_PT)])
    pltpu.sync_copy(r_hbm, rvm)
    plsc.subcore_barrier()
    wbase = wid * EPW

    def body(u, carry):
        base = wbase + u * 128
        pltpu.sync_copy(srcp.at[pl.ds(base, 128)], sidx)
        pltpu.sync_copy(dstp.at[pl.ds(base, 128)], didx)
        pltpu.sync_copy(e_hbm.at[pl.ds(base, 128)], ebuf)
        for k in range(8):
            d16 = didx[pl.ds(k * 16, 16)]
            rd = plsc.load_gather(rvm, [d16])
            abuf[pl.ds(k * 16, 16)] = ebuf[pl.ds(k * 16, 16)] * rd
        pltpu.async_copy(h_hbm.at[sidx], rows, sem).wait()

        def scale(j, carry):
            aj = _sload(abuf, j)
            for cc in range(8):
                sl = pl.ds(cc * 16, 16)
                rows[j, sl] = rows[j, sl] * aj
            return carry

        lax.fori_loop(0, 128, scale, 0)
        pltpu.sync_copy(rows, acc.at[didx], add=True)
        return carry

    lax.fori_loop(0, UNITS, body, 0)
    plsc.subcore_barrier()
    pltpu.sync_copy(acc.at[pl.ds(r0, ROWS_PT)], out.at[cid, pl.ds(r0, ROWS_PT)])


def _wsegsum_sc(hp, srcp, dstp, eattn, rden):
    zin = jnp.zeros((NPAD, D), jnp.float32)
    mesh = plsc.VectorSubcoreMesh(core_axis_name="c", subcore_axis_name="s",
                                  num_cores=SC_CORES, num_subcores=SC_TILES)
    return pl.kernel(
        _wsegsum_body,
        out_type=jax.ShapeDtypeStruct((SC_CORES, NPAD, D), jnp.float32),
        mesh=mesh,
        compiler_params=pltpu.CompilerParams(needs_layout_passes=False),
        scratch_types=[
            pltpu.VMEM_SHARED((NPAD, D), jnp.float32),
            pltpu.VMEM((NPAD,), jnp.float32),
            pltpu.VMEM((128,), jnp.int32),
            pltpu.VMEM((128,), jnp.int32),
            pltpu.VMEM((128,), jnp.float32),
            pltpu.VMEM((144,), jnp.float32),
            pltpu.VMEM((128, D), jnp.float32),
            pltpu.SemaphoreType.DMA,
        ],
    )(hp, srcp, dstp, eattn, rden, zin)


def _gin_mlp_body(scale, h_ref, a0_ref, a1_ref, w1_ref, b1_ref, w2_ref, b2_ref, o_ref):
    z = h_ref[...] + scale * (a0_ref[...] + a1_ref[...])
    z = jnp.maximum(jnp.dot(z, w1_ref[...], preferred_element_type=jnp.float32) + b1_ref[...], 0.0)
    o_ref[...] = jnp.maximum(jnp.dot(z, w2_ref[...], preferred_element_type=jnp.float32) + b2_ref[...], 0.0)


def _matcols_body(h_ref, w_ref, b_ref, o_ref):
    o_ref[...] = jnp.dot(h_ref[...], w_ref[...],
                         preferred_element_type=jnp.float32) + b_ref[...]


def _matcols(h, wcols, brow):
    row_spec = pl.BlockSpec((BLK, D), lambda i: (i, 0))
    return pl.pallas_call(
        _matcols_body,
        grid=(NPAD // BLK,),
        in_specs=[row_spec, pl.BlockSpec((D, 128), lambda i: (0, 0)),
                  pl.BlockSpec((1, 128), lambda i: (0, 0))],
        out_specs=pl.BlockSpec((BLK, 128), lambda i: (i, 0)),
        out_shape=jax.ShapeDtypeStruct((NPAD, 128), jnp.float32),
    )(h, wcols, brow)


def _poolred_body(dp_ref, gp_ref, u_ref, v_ref, m_ref, rden_ref, ws_ref, deg_ref):
    es = jnp.exp(_leaky(u_ref[...] + v_ref[...]) - m_ref[...])
    den = jnp.sum(dp_ref[...], axis=0, keepdims=True) + es
    rden = 1.0 / (den + 1e-16)
    rden_ref[...] = rden
    ws_ref[...] = es * rden
    deg_ref[...] = jnp.sum(gp_ref[...], axis=0, keepdims=True)


def _poolred(den_parts, deg_parts, u, v, m):
    cspec = pl.BlockSpec((1, 128), lambda i: (0, i))
    pspec = pl.BlockSpec((NWORK, 128), lambda i: (0, i))
    out1d = jax.ShapeDtypeStruct((1, NPAD), jnp.float32)
    return pl.pallas_call(
        _poolred_body,
        grid=(NPAD // 128,),
        in_specs=[pspec, pspec, cspec, cspec, cspec],
        out_specs=[cspec, cspec, cspec],
        out_shape=[out1d, out1d, out1d],
    )(den_parts, deg_parts, u.reshape(1, NPAD), v.reshape(1, NPAD),
      m.reshape(1, NPAD))


def _xcabc_body(p0_ref, p1_ref, h_ref, ws_ref, w_ref, b_ref, xc_ref, abc_ref):
    xc = p0_ref[...] + p1_ref[...] + h_ref[...] * ws_ref[...]
    xc_ref[...] = xc
    abc_ref[...] = jnp.dot(xc, w_ref[...],
                           preferred_element_type=jnp.float32) + b_ref[...]


def _xcabc(p0, p1, hp, ws_col, wcols, brow):
    row_spec = pl.BlockSpec((BLK, D), lambda i: (i, 0))
    return pl.pallas_call(
        _xcabc_body,
        grid=(NPAD // BLK,),
        in_specs=[row_spec, row_spec, row_spec,
                  pl.BlockSpec((BLK, 1), lambda i: (i, 0)),
                  pl.BlockSpec((D, 128), lambda i: (0, 0)),
                  pl.BlockSpec((1, 128), lambda i: (0, 0))],
        out_specs=[row_spec, pl.BlockSpec((BLK, 128), lambda i: (i, 0))],
        out_shape=[jax.ShapeDtypeStruct((NPAD, D), jnp.float32),
                   jax.ShapeDtypeStruct((NPAD, 128), jnp.float32)],
    )(p0, p1, hp, ws_col, wcols, brow)


def _gin_mlp(h, a0, a1, w1, b1, w2, b2, scale=1.0):
    grid = (NPAD // BLK,)
    row_spec = pl.BlockSpec((BLK, D), lambda i: (i, 0))
    w_spec = pl.BlockSpec((D, D), lambda i: (0, 0))
    b_spec = pl.BlockSpec((1, D), lambda i: (0, 0))
    return pl.pallas_call(
        functools.partial(_gin_mlp_body, scale),
        grid=grid,
        in_specs=[row_spec, row_spec, row_spec, w_spec, b_spec, w_spec, b_spec],
        out_specs=row_spec,
        out_shape=jax.ShapeDtypeStruct((NPAD, D), jnp.float32),
    )(h, a0, a1, w1, b1.reshape(1, D), w2, b2.reshape(1, D))


def _pad_rows(x):
    return jnp.pad(x, ((0, NPAD - N), (0, 0)))


def kernel(x, enc_W1, enc_b1, enc_W2, enc_b2, enc_W3, enc_b3, enc_W4, enc_b4,
           pool_lin_W, pool_lin_b, pool_att_W, pool_att_b,
           score_W1, score_b1, score_W2, score_W3,
           gnn_W1, gnn_b1, gnn_W2, gnn_b2, gnn_W3, gnn_b3, gnn_W4, gnn_b4,
           cls_W, cls_b, edge_index, batch):
    src, dst = edge_index[0], edge_index[1]
    epad = jnp.full((EPAD - src.shape[0],), N, jnp.int32)
    srcp = jnp.concatenate([src, epad])
    dstp = jnp.concatenate([dst, epad])

    def gin(hp, dp, w1, b1, w2, b2, scale=1.0):
        p = _segsum_sc(hp, srcp, dp)
        return _gin_mlp(hp, p[0], p[1], w1, b1, w2, b2, scale)

    hp = gin(_pad_rows(x), dstp, enc_W1, enc_b1, enc_W2, enc_b2)
    hp = gin(hp, dstp, enc_W3, enc_b3, enc_W4, enc_b4)
    h = hp[:N]

    a1 = pool_att_W[:D, 0]
    a2 = pool_att_W[D:, 0]
    wa = pool_lin_W @ a1
    vconst = pool_lin_b @ a1 + pool_att_b[0]
    wv = jnp.zeros((D, 128), jnp.float32).at[:, 0].set(a2)
    bv = jnp.zeros((1, 128), jnp.float32).at[0, 0].set(vconst)
    v = _matcols(hp, wv, bv)[:, 0]
    u, m = _poolmax_sc(hp, src, dst, v, wa)
    den_parts, deg_parts, e_arr = _scalsum_sc(src, dst, u, v, m)
    rden2, ws2, deg2 = _poolred(den_parts, deg_parts, u, v, m)
    eattn = jnp.concatenate([e_arr, jnp.zeros((EPAD - E,), jnp.float32)])
    p = _wsegsum_sc(hp, srcp, dstp, eattn, rden2.reshape(NPAD))
    wsc = (jnp.zeros((D, 128), jnp.float32)
           .at[:, 0].set(score_W1[:, 0])
           .at[:, 1].set(score_W2[:, 0])
           .at[:, 2].set(score_W3[:, 0]))
    bsc = jnp.zeros((1, 128), jnp.float32).at[0, 0].set(score_b1[0])
    xcp, abc = _xcabc(p[0], p[1], hp, ws2.reshape(NPAD, 1), wsc, bsc)
    c0, aa, b3 = abc[:N, 0], abc[:N, 1], abc[:N, 2]
    s_sum = jax.ops.segment_sum(b3[src], dst, num_segments=N)
    fit = c0 + aa - b3 + deg2.reshape(NPAD)[:N] * aa - s_sum
    fitness = jax.nn.sigmoid(fit)
    topv, perm = jax.lax.top_k(fitness, KSEL)
    kept = jnp.zeros((N,), jnp.float32).at[perm].set(1.0)
    w = kept * fitness
    px = xcp[:N] * w[:, None]
    em = kept[src] * kept[dst]
    c = float(1.0 / (1.0 + math.exp(-1.0)))
    dstm = jnp.concatenate([jnp.where(em > 0.5, dst, N).astype(jnp.int32), epad])

    gp = gin(_pad_rows(px), dstm, gnn_W1, gnn_b1, gnn_W2, gnn_b2, scale=c)
    gp = gin(gp, dstm, gnn_W3, gnn_b3, gnn_W4, gnn_b4, scale=c)
    g = gp[:N]

    sums = jax.ops.segment_sum(g * kept[:, None], batch, num_segments=NG)
    cnt = jax.ops.segment_sum(kept, batch, num_segments=NG)
    readout = sums / jnp.maximum(cnt, 1.0)[:, None]
    return readout @ cls_W + cls_b


# speedup vs baseline: 2.4690x; 1.0171x over previous
import functools
import math

import jax
import jax.numpy as jnp
from jax import lax
from jax.experimental import pallas as pl
from jax.experimental.pallas import tpu as pltpu
from jax.experimental.pallas import tpu_sc as plsc

N = 10000
D = 128
NG = 128
NC = 10
KSEL = N // 2

BLK = 128
NPAD = 10240

SC_CORES = 2
SC_TILES = 16
NWORK = SC_CORES * SC_TILES
EPAD = 327680
EPW = EPAD // NWORK
UNITS = EPW // 128
ROWS_PT = NPAD // SC_TILES
E = 320000
RNG = NPAD // NWORK
SCAP = 16352
CH = 2048
NCH = 157


def _segsum_sc_body(h_hbm, srcp, dstp, zin, out,
                    acc, sidx0, didx0, rows0, sidx1, didx1, rows1, sem):
    cid = lax.axis_index("c")
    sid = lax.axis_index("s")
    wid = sid * SC_CORES + cid
    r0 = sid * ROWS_PT
    pltpu.sync_copy(zin.at[pl.ds(r0, ROWS_PT)], acc.at[pl.ds(r0, ROWS_PT)])
    plsc.subcore_barrier()
    wbase = wid * EPW
    bufs = ((sidx0, didx0, rows0), (sidx1, didx1, rows1))

    pltpu.sync_copy(srcp.at[pl.ds(wbase, 128)], sidx0)
    pltpu.sync_copy(dstp.at[pl.ds(wbase, 128)], didx0)
    pltpu.async_copy(h_hbm.at[sidx0], rows0, sem)

    def pair(g, carry):
        for b in range(2):
            u = g * 2 + b
            sb, db, rb = bufs[b]
            sn, dn, rn = bufs[1 - b]
            pltpu.make_async_copy(h_hbm.at[sb], rb, sem).wait()

            @pl.when(u + 1 < UNITS)
            def _():
                nbase = wbase + (u + 1) * 128
                pltpu.sync_copy(srcp.at[pl.ds(nbase, 128)], sn)
                pltpu.sync_copy(dstp.at[pl.ds(nbase, 128)], dn)
                pltpu.async_copy(h_hbm.at[sn], rn, sem)

            pltpu.sync_copy(rb, acc.at[db], add=True)
        return carry

    lax.fori_loop(0, UNITS // 2, pair, 0)
    plsc.subcore_barrier()
    pltpu.sync_copy(acc.at[pl.ds(r0, ROWS_PT)], out.at[cid, pl.ds(r0, ROWS_PT)])


def _segsum_sc(hp, srcp, dstp):
    zin = jnp.zeros((NPAD, D), jnp.float32)
    mesh = plsc.VectorSubcoreMesh(core_axis_name="c", subcore_axis_name="s",
                                  num_cores=SC_CORES, num_subcores=SC_TILES)
    return pl.kernel(
        _segsum_sc_body,
        out_type=jax.ShapeDtypeStruct((SC_CORES, NPAD, D), jnp.float32),
        mesh=mesh,
        scratch_types=[
            pltpu.VMEM_SHARED((NPAD, D), jnp.float32),
            pltpu.VMEM((128,), jnp.int32),
            pltpu.VMEM((128,), jnp.int32),
            pltpu.VMEM((128, D), jnp.float32),
            pltpu.VMEM((128,), jnp.int32),
            pltpu.VMEM((128,), jnp.int32),
            pltpu.VMEM((128, D), jnp.float32),
            pltpu.SemaphoreType.DMA,
        ],
    )(hp, srcp, dstp, zin)


def _leaky(t):
    return jnp.where(t > 0, t, 0.2 * t)


def _sload(ref, i):
    return ref[pl.ds(i, 16)][0]


def _sstore(ref, i, val):
    lane0 = lax.iota(jnp.int32, 16) == 0
    plsc.store_scatter(ref, [jnp.full((16,), i, jnp.int32)],
                       jnp.full((16,), val, ref.dtype), mask=lane0)


def _poolmax_body(h_hbm, src_h, dst_h, v_hbm, wa_hbm, u_out, m_out,
                  acc, ss, sd, svm, dvm, v_vm, wa_vm, u_vm, m_vm, rows, sem):
    cid = lax.axis_index("c")
    sid = lax.axis_index("s")
    wid = sid * SC_CORES + cid
    lo = wid * RNG
    pltpu.sync_copy(h_hbm.at[pl.ds(lo, RNG)], acc.at[pl.ds(0, RNG)])
    pltpu.sync_copy(v_hbm, v_vm.at[pl.ds(0, NPAD)])
    pltpu.sync_copy(wa_hbm, wa_vm)
    dummy_s = jnp.full((16,), N, jnp.int32)
    dummy_d = jnp.full((16,), RNG, jnp.int32)

    def prefill(j, carry):
        ss[pl.ds(j * 16, 16)] = dummy_s
        sd[pl.ds(j * 16, 16)] = dummy_d
        return carry

    lax.fori_loop(0, (SCAP + 160) // 16, prefill, 0)

    tail = E - (NCH - 1) * CH

    def chunk_body(ch, pos):
        base = ch * CH
        pltpu.sync_copy(src_h.at[pl.ds(base, tail)], svm.at[pl.ds(0, tail)])
        pltpu.sync_copy(dst_h.at[pl.ds(base, tail)], dvm.at[pl.ds(0, tail)])

        @pl.when(ch < NCH - 1)
        def _():
            pltpu.sync_copy(src_h.at[pl.ds(base + tail, CH - tail)],
                            svm.at[pl.ds(tail, CH - tail)])
            pltpu.sync_copy(dst_h.at[pl.ds(base + tail, CH - tail)],
                            dvm.at[pl.ds(tail, CH - tail)])
        nsv = jnp.where(ch < NCH - 1, CH // 16, tail // 16)

        def sub_body(k, pos):
            s16 = svm[pl.ds(k * 16, 16)]
            d16 = dvm[pl.ds(k * 16, 16)]
            msk = (d16 >= lo) & (d16 < lo + RNG)
            cnt = jnp.sum(msk.astype(jnp.int32))
            p = jnp.minimum(pos, SCAP)
            plsc.store_compressed(ss.at[pl.ds(p, 16)], s16, mask=msk)
            plsc.store_compressed(sd.at[pl.ds(p, 16)], d16 - lo, mask=msk)
            return pos + cnt

        return lax.fori_loop(0, nsv, sub_body, pos)

    pos = lax.fori_loop(0, NCH, chunk_body, jnp.int32(0))
    pos = jnp.minimum(pos, SCAP)

    def flush(f, carry):
        pltpu.async_copy(h_hbm.at[ss.at[pl.ds(f * 128, 128)]], rows, sem).wait()

        def upd(j, carry):
            dl = _sload(sd, f * 128 + j)
            for cc in range(8):
                sl = pl.ds(cc * 16, 16)
                acc[dl, sl] = jnp.maximum(acc[dl, sl], rows[j, sl])
            return carry

        return lax.fori_loop(0, 128, upd, carry)

    lax.fori_loop(0, (pos + 127) // 128, flush, 0)

    def udot(r, carry):
        t = jnp.zeros((16,), jnp.float32)
        for cc in range(8):
            sl = pl.ds(cc * 16, 16)
            t = t + acc[r, sl] * wa_vm[sl]
        uu = jnp.sum(t)
        _sstore(u_vm, r, uu)
        _sstore(m_vm, r, _leaky(uu + _sload(v_vm, lo + r)))
        return carry

    lax.fori_loop(0, RNG, udot, 0)

    def mupd(j, carry):
        dl = _sload(sd, j)
        sc = _leaky(_sload(u_vm, dl) + _sload(v_vm, _sload(ss, j)))
        _sstore(m_vm, dl, jnp.maximum(_sload(m_vm, dl), sc))
        return carry

    lax.fori_loop(0, pos, mupd, 0)
    pltpu.sync_copy(u_vm.at[pl.ds(0, RNG)], u_out.at[pl.ds(lo, RNG)])
    pltpu.sync_copy(m_vm.at[pl.ds(0, RNG)], m_out.at[pl.ds(lo, RNG)])


def _poolmax_sc(hp, src_e, dst_e, v, wa):
    mesh = plsc.VectorSubcoreMesh(core_axis_name="c", subcore_axis_name="s",
                                  num_cores=SC_CORES, num_subcores=SC_TILES)
    return pl.kernel(
        _poolmax_body,
        out_type=(jax.ShapeDtypeStruct((NPAD,), jnp.float32),
                  jax.ShapeDtypeStruct((NPAD,), jnp.float32)),
        mesh=mesh,
        compiler_params=pltpu.CompilerParams(needs_layout_passes=False),
        scratch_types=[
            pltpu.VMEM((RNG + 8, D), jnp.float32),
            pltpu.VMEM((SCAP + 160,), jnp.int32),
            pltpu.VMEM((SCAP + 160,), jnp.int32),
            pltpu.VMEM((CH,), jnp.int32),
            pltpu.VMEM((CH,), jnp.int32),
            pltpu.VMEM((NPAD + 16,), jnp.float32),
            pltpu.VMEM((D,), jnp.float32),
            pltpu.VMEM((RNG + 16,), jnp.float32),
            pltpu.VMEM((RNG + 16,), jnp.float32),
            pltpu.VMEM((128, D), jnp.float32),
            pltpu.SemaphoreType.DMA,
        ],
    )(hp, src_e, dst_e, v, wa)


def _scalsum_body(src_h, dst_h, u_hbm, v_hbm, m_hbm, den_out, deg_out, e_out,
                  uvm, vvm, mvm, dacc, gacc, svm, dvm, ebuf):
    cid = lax.axis_index("c")
    sid = lax.axis_index("s")
    wid = sid * SC_CORES + cid
    pltpu.sync_copy(u_hbm, uvm)
    pltpu.sync_copy(v_hbm, vvm)
    pltpu.sync_copy(m_hbm, mvm)
    zz = jnp.zeros((16,), jnp.float32)

    def zinit(j, carry):
        dacc[pl.ds(j * 16, 16)] = zz
        gacc[pl.ds(j * 16, 16)] = zz
        return carry

    lax.fori_loop(0, NPAD // 16, zinit, 0)
    epw = E // NWORK
    ones = jnp.ones((16,), jnp.float32)

    def chunk_body(ch, carry):
        base = wid * epw + ch * CH
        csz = jnp.minimum(epw - ch * CH, CH)
        pltpu.sync_copy(src_h.at[pl.ds(base, 1808)], svm.at[pl.ds(0, 1808)])
        pltpu.sync_copy(dst_h.at[pl.ds(base, 1808)], dvm.at[pl.ds(0, 1808)])

        @pl.when(csz == CH)
        def _():
            pltpu.sync_copy(src_h.at[pl.ds(base + 1808, CH - 1808)],
                            svm.at[pl.ds(1808, CH - 1808)])
            pltpu.sync_copy(dst_h.at[pl.ds(base + 1808, CH - 1808)],
                            dvm.at[pl.ds(1808, CH - 1808)])

        def sub_body(k, carry):
            s16 = svm[pl.ds(k * 16, 16)]
            d16 = dvm[pl.ds(k * 16, 16)]
            ud = plsc.load_gather(uvm, [d16])
            vs = plsc.load_gather(vvm, [s16])
            md = plsc.load_gather(mvm, [d16])
            e = jnp.exp(_leaky(ud + vs) - md)
            ebuf[pl.ds(k * 16, 16)] = e
            plsc.addupdate_scatter(dacc, [d16], e)
            plsc.addupdate_scatter(gacc, [d16], ones)
            return carry

        lax.fori_loop(0, csz // 16, sub_body, carry)
        pltpu.sync_copy(ebuf.at[pl.ds(0, 1808)], e_out.at[pl.ds(base, 1808)])

        @pl.when(csz == CH)
        def _():
            pltpu.sync_copy(ebuf.at[pl.ds(1808, CH - 1808)],
                            e_out.at[pl.ds(base + 1808, CH - 1808)])
        return carry

    lax.fori_loop(0, (epw + CH - 1) // CH, chunk_body, 0)
    pltpu.sync_copy(dacc, den_out.at[wid])
    pltpu.sync_copy(gacc, deg_out.at[wid])


def _scalsum_sc(src_e, dst_e, u, v, m):
    mesh = plsc.VectorSubcoreMesh(core_axis_name="c", subcore_axis_name="s",
                                  num_cores=SC_CORES, num_subcores=SC_TILES)
    return pl.kernel(
        _scalsum_body,
        out_type=(jax.ShapeDtypeStruct((NWORK, NPAD), jnp.float32),
                  jax.ShapeDtypeStruct((NWORK, NPAD), jnp.float32),
                  jax.ShapeDtypeStruct((E,), jnp.float32)),
        mesh=mesh,
        compiler_params=pltpu.CompilerParams(needs_layout_passes=False),
        scratch_types=[
            pltpu.VMEM((NPAD,), jnp.float32),
            pltpu.VMEM((NPAD,), jnp.float32),
            pltpu.VMEM((NPAD,), jnp.float32),
            pltpu.VMEM((NPAD,), jnp.float32),
            pltpu.VMEM((NPAD,), jnp.float32),
            pltpu.VMEM((CH,), jnp.int32),
            pltpu.VMEM((CH,), jnp.int32),
            pltpu.VMEM((CH,), jnp.float32),
        ],
    )(src_e, dst_e, u, v, m)


def _wsegsum_body(h_hbm, srcp, dstp, e_hbm, r_hbm, zin, out,
                  acc, rvm, sidx, didx, ebuf, abuf, rows, sem):
    cid = lax.axis_index("c")
    sid = lax.axis_index("s")
    wid = sid * SC_CORES + cid
    r0 = sid * ROWS_PT
    pltpu.sync_copy(zin.at[pl.ds(r0, ROWS_PT)], acc.at[pl.ds(r0, ROWS_PT)])
    pltpu.sync_copy(r_hbm, rvm)
    plsc.subcore_barrier()
    wbase = wid * EPW

    def body(u, carry):
        base = wbase + u * 128
        pltpu.sync_copy(srcp.at[pl.ds(base, 128)], sidx)
        pltpu.sync_copy(dstp.at[pl.ds(base, 128)], didx)
        pltpu.sync_copy(e_hbm.at[pl.ds(base, 128)], ebuf)
        for k in range(8):
            d16 = didx[pl.ds(k * 16, 16)]
            rd = plsc.load_gather(rvm, [d16])
            abuf[pl.ds(k * 16, 16)] = ebuf[pl.ds(k * 16, 16)] * rd
        pltpu.async_copy(h_hbm.at[sidx], rows, sem).wait()

        def scale(j, carry):
            aj = _sload(abuf, j)
            for cc in range(8):
                sl = pl.ds(cc * 16, 16)
                rows[j, sl] = rows[j, sl] * aj
            return carry

        lax.fori_loop(0, 128, scale, 0)
        pltpu.sync_copy(rows, acc.at[didx], add=True)
        return carry

    lax.fori_loop(0, UNITS, body, 0)
    plsc.subcore_barrier()
    pltpu.sync_copy(acc.at[pl.ds(r0, ROWS_PT)], out.at[cid, pl.ds(r0, ROWS_PT)])


def _wsegsum_sc(hp, srcp, dstp, eattn, rden):
    zin = jnp.zeros((NPAD, D), jnp.float32)
    mesh = plsc.VectorSubcoreMesh(core_axis_name="c", subcore_axis_name="s",
                                  num_cores=SC_CORES, num_subcores=SC_TILES)
    return pl.kernel(
        _wsegsum_body,
        out_type=jax.ShapeDtypeStruct((SC_CORES, NPAD, D), jnp.float32),
        mesh=mesh,
        compiler_params=pltpu.CompilerParams(needs_layout_passes=False),
        scratch_types=[
            pltpu.VMEM_SHARED((NPAD, D), jnp.float32),
            pltpu.VMEM((NPAD,), jnp.float32),
            pltpu.VMEM((128,), jnp.int32),
            pltpu.VMEM((128,), jnp.int32),
            pltpu.VMEM((128,), jnp.float32),
            pltpu.VMEM((144,), jnp.float32),
            pltpu.VMEM((128, D), jnp.float32),
            pltpu.SemaphoreType.DMA,
        ],
    )(hp, srcp, dstp, eattn, rden, zin)


def _gin_mlp_body(scale, h_ref, a0_ref, a1_ref, w1_ref, b1_ref, w2_ref, b2_ref, o_ref):
    z = h_ref[...] + scale * (a0_ref[...] + a1_ref[...])
    z = jnp.maximum(jnp.dot(z, w1_ref[...], preferred_element_type=jnp.float32) + b1_ref[...], 0.0)
    o_ref[...] = jnp.maximum(jnp.dot(z, w2_ref[...], preferred_element_type=jnp.float32) + b2_ref[...], 0.0)


def _matcols_body(h_ref, w_ref, b_ref, o_ref):
    o_ref[...] = jnp.dot(h_ref[...], w_ref[...],
                         preferred_element_type=jnp.float32) + b_ref[...]


def _matcols(h, wcols, brow):
    row_spec = pl.BlockSpec((BLK, D), lambda i: (i, 0))
    return pl.pallas_call(
        _matcols_body,
        grid=(NPAD // BLK,),
        in_specs=[row_spec, pl.BlockSpec((D, 128), lambda i: (0, 0)),
                  pl.BlockSpec((1, 128), lambda i: (0, 0))],
        out_specs=pl.BlockSpec((BLK, 128), lambda i: (i, 0)),
        out_shape=jax.ShapeDtypeStruct((NPAD, 128), jnp.float32),
    )(h, wcols, brow)


def _poolred_body(dp_ref, gp_ref, u_ref, v_ref, m_ref, rden_ref, ws_ref, deg_ref):
    es = jnp.exp(_leaky(u_ref[...] + v_ref[...]) - m_ref[...])
    den = jnp.sum(dp_ref[...], axis=0, keepdims=True) + es
    rden = 1.0 / (den + 1e-16)
    rden_ref[...] = rden
    ws_ref[...] = es * rden
    deg_ref[...] = jnp.sum(gp_ref[...], axis=0, keepdims=True)


def _poolred(den_parts, deg_parts, u, v, m):
    cspec = pl.BlockSpec((1, 128), lambda i: (0, i))
    pspec = pl.BlockSpec((NWORK, 128), lambda i: (0, i))
    out1d = jax.ShapeDtypeStruct((1, NPAD), jnp.float32)
    return pl.pallas_call(
        _poolred_body,
        grid=(NPAD // 128,),
        in_specs=[pspec, pspec, cspec, cspec, cspec],
        out_specs=[cspec, cspec, cspec],
        out_shape=[out1d, out1d, out1d],
    )(den_parts, deg_parts, u.reshape(1, NPAD), v.reshape(1, NPAD),
      m.reshape(1, NPAD))


def _xcabc_body(p0_ref, p1_ref, h_ref, ws_ref, w_ref, b_ref, xc_ref, abc_ref):
    xc = p0_ref[...] + p1_ref[...] + h_ref[...] * ws_ref[...]
    xc_ref[...] = xc
    abc_ref[...] = jnp.dot(xc, w_ref[...],
                           preferred_element_type=jnp.float32) + b_ref[...]


def _xcabc(p0, p1, hp, ws_col, wcols, brow):
    row_spec = pl.BlockSpec((BLK, D), lambda i: (i, 0))
    return pl.pallas_call(
        _xcabc_body,
        grid=(NPAD // BLK,),
        in_specs=[row_spec, row_spec, row_spec,
                  pl.BlockSpec((BLK, 1), lambda i: (i, 0)),
                  pl.BlockSpec((D, 128), lambda i: (0, 0)),
                  pl.BlockSpec((1, 128), lambda i: (0, 0))],
        out_specs=[row_spec, pl.BlockSpec((BLK, 128), lambda i: (i, 0))],
        out_shape=[jax.ShapeDtypeStruct((NPAD, D), jnp.float32),
                   jax.ShapeDtypeStruct((NPAD, 128), jnp.float32)],
    )(p0, p1, hp, ws_col, wcols, brow)


def _gin_mlp(h, a0, a1, w1, b1, w2, b2, scale=1.0):
    grid = (NPAD // BLK,)
    row_spec = pl.BlockSpec((BLK, D), lambda i: (i, 0))
    w_spec = pl.BlockSpec((D, D), lambda i: (0, 0))
    b_spec = pl.BlockSpec((1, D), lambda i: (0, 0))
    return pl.pallas_call(
        functools.partial(_gin_mlp_body, scale),
        grid=grid,
        in_specs=[row_spec, row_spec, row_spec, w_spec, b_spec, w_spec, b_spec],
        out_specs=row_spec,
        out_shape=jax.ShapeDtypeStruct((NPAD, D), jnp.float32),
    )(h, a0, a1, w1, b1.reshape(1, D), w2, b2.reshape(1, D))


def _pad_rows(x):
    return jnp.pad(x, ((0, NPAD - N), (0, 0)))


def kernel(x, enc_W1, enc_b1, enc_W2, enc_b2, enc_W3, enc_b3, enc_W4, enc_b4,
           pool_lin_W, pool_lin_b, pool_att_W, pool_att_b,
           score_W1, score_b1, score_W2, score_W3,
           gnn_W1, gnn_b1, gnn_W2, gnn_b2, gnn_W3, gnn_b3, gnn_W4, gnn_b4,
           cls_W, cls_b, edge_index, batch):
    src, dst = edge_index[0], edge_index[1]
    epad = jnp.full((EPAD - src.shape[0],), N, jnp.int32)
    srcp = jnp.concatenate([src, epad])
    dstp = jnp.concatenate([dst, epad])

    def gin(hp, dp, w1, b1, w2, b2, scale=1.0):
        p = _segsum_sc(hp, srcp, dp)
        return _gin_mlp(hp, p[0], p[1], w1, b1, w2, b2, scale)

    hp = gin(_pad_rows(x), dstp, enc_W1, enc_b1, enc_W2, enc_b2)
    hp = gin(hp, dstp, enc_W3, enc_b3, enc_W4, enc_b4)
    h = hp[:N]

    a1 = pool_att_W[:D, 0]
    a2 = pool_att_W[D:, 0]
    wa = pool_lin_W @ a1
    vconst = pool_lin_b @ a1 + pool_att_b[0]
    wv = jnp.zeros((D, 128), jnp.float32).at[:, 0].set(a2)
    bv = jnp.zeros((1, 128), jnp.float32).at[0, 0].set(vconst)
    v = _matcols(hp, wv, bv)[:, 0]
    u, m = _poolmax_sc(hp, src, dst, v, wa)
    den_parts, deg_parts, e_arr = _scalsum_sc(src, dst, u, v, m)
    rden2, ws2, deg2 = _poolred(den_parts, deg_parts, u, v, m)
    eattn = jnp.concatenate([e_arr, jnp.zeros((EPAD - E,), jnp.float32)])
    p = _wsegsum_sc(hp, srcp, dstp, eattn, rden2.reshape(NPAD))
    wsc = (jnp.zeros((D, 128), jnp.float32)
           .at[:, 0].set(score_W1[:, 0])
           .at[:, 1].set(score_W2[:, 0])
           .at[:, 2].set(score_W3[:, 0]))
    bsc = jnp.zeros((1, 128), jnp.float32).at[0, 0].set(score_b1[0])
    xcp, abc = _xcabc(p[0], p[1], hp, ws2.reshape(NPAD, 1), wsc, bsc)
    c0, aa, b3 = abc[:N, 0], abc[:N, 1], abc[:N, 2]
    s_sum = jax.ops.segment_sum(b3[src], dst, num_segments=N)
    fit = c0 + aa - b3 + deg2.reshape(NPAD)[:N] * aa - s_sum
    fitness = jax.nn.sigmoid(fit)
    topv, perm = jax.lax.top_k(fitness, KSEL)
    kept = jnp.zeros((N,), jnp.float32).at[perm].set(1.0)
    w = kept * fitness
    px = xcp[:N] * w[:, None]
    em = kept[src] * kept[dst]
    c = float(1.0 / (1.0 + math.exp(-1.0)))
    dstm = jnp.concatenate([jnp.where(em > 0.5, dst, N).astype(jnp.int32), epad])

    gp = gin(_pad_rows(px), dstm, gnn_W1, gnn_b1, gnn_W2, gnn_b2, scale=c)
    gp = gin(gp, dstm, gnn_W3, gnn_b3, gnn_W4, gnn_b4, scale=c)
    g = gp[:N]

    sums = jax.ops.segment_sum(g * kept[:, None], batch, num_segments=NG)
    cnt = jax.ops.segment_sum(kept, batch, num_segments=NG)
    readout = sums / jnp.maximum(cnt, 1.0)[:, None]
    return readout @ cls_W + cls_b


# speedup vs baseline: 2.6935x; 1.0909x over previous
import functools
import math

import jax
import jax.numpy as jnp
from jax import lax
from jax.experimental import pallas as pl
from jax.experimental.pallas import tpu as pltpu
from jax.experimental.pallas import tpu_sc as plsc

N = 10000
D = 128
NG = 128
NC = 10
KSEL = N // 2

BLK = 128
NPAD = 10240

SC_CORES = 2
SC_TILES = 16
NWORK = SC_CORES * SC_TILES
EPAD = 327680
EPW = EPAD // NWORK
UNITS = EPW // 128
ROWS_PT = NPAD // SC_TILES
E = 320000
RNG = NPAD // NWORK
SCAP = 16352
CH = 2048
NCH = 157


def _segsum_sc_body(h_hbm, srcp, dstp, zin, out,
                    acc, sidx0, didx0, rows0, sidx1, didx1, rows1, sem):
    cid = lax.axis_index("c")
    sid = lax.axis_index("s")
    wid = sid * SC_CORES + cid
    r0 = sid * ROWS_PT
    pltpu.sync_copy(zin.at[pl.ds(r0, ROWS_PT)], acc.at[pl.ds(r0, ROWS_PT)])
    plsc.subcore_barrier()
    wbase = wid * EPW
    bufs = ((sidx0, didx0, rows0), (sidx1, didx1, rows1))

    pltpu.sync_copy(srcp.at[pl.ds(wbase, 128)], sidx0)
    pltpu.sync_copy(dstp.at[pl.ds(wbase, 128)], didx0)
    pltpu.async_copy(h_hbm.at[sidx0], rows0, sem)

    def pair(g, carry):
        for b in range(2):
            u = g * 2 + b
            sb, db, rb = bufs[b]
            sn, dn, rn = bufs[1 - b]
            pltpu.make_async_copy(h_hbm.at[sb], rb, sem).wait()

            @pl.when(u + 1 < UNITS)
            def _():
                nbase = wbase + (u + 1) * 128
                pltpu.sync_copy(srcp.at[pl.ds(nbase, 128)], sn)
                pltpu.sync_copy(dstp.at[pl.ds(nbase, 128)], dn)
                pltpu.async_copy(h_hbm.at[sn], rn, sem)

            pltpu.sync_copy(rb, acc.at[db], add=True)
        return carry

    lax.fori_loop(0, UNITS // 2, pair, 0)
    plsc.subcore_barrier()
    pltpu.sync_copy(acc.at[pl.ds(r0, ROWS_PT)], out.at[cid, pl.ds(r0, ROWS_PT)])


def _segsum_sc(hp, srcp, dstp):
    zin = jnp.zeros((NPAD, D), jnp.float32)
    mesh = plsc.VectorSubcoreMesh(core_axis_name="c", subcore_axis_name="s",
                                  num_cores=SC_CORES, num_subcores=SC_TILES)
    return pl.kernel(
        _segsum_sc_body,
        out_type=jax.ShapeDtypeStruct((SC_CORES, NPAD, D), jnp.float32),
        mesh=mesh,
        scratch_types=[
            pltpu.VMEM_SHARED((NPAD, D), jnp.float32),
            pltpu.VMEM((128,), jnp.int32),
            pltpu.VMEM((128,), jnp.int32),
            pltpu.VMEM((128, D), jnp.float32),
            pltpu.VMEM((128,), jnp.int32),
            pltpu.VMEM((128,), jnp.int32),
            pltpu.VMEM((128, D), jnp.float32),
            pltpu.SemaphoreType.DMA,
        ],
    )(hp, srcp, dstp, zin)


def _leaky(t):
    return jnp.where(t > 0, t, 0.2 * t)


def _sload(ref, i):
    return ref[pl.ds(i, 16)][0]


def _sstore(ref, i, val):
    lane0 = lax.iota(jnp.int32, 16) == 0
    plsc.store_scatter(ref, [jnp.full((16,), i, jnp.int32)],
                       jnp.full((16,), val, ref.dtype), mask=lane0)


def _poolmax_body(h_hbm, src_h, dst_h, v_hbm, wa_hbm, u_out, m_out,
                  acc, ss, sd, svm, dvm, v_vm, wa_vm, u_vm, m_vm, rows, sem):
    cid = lax.axis_index("c")
    sid = lax.axis_index("s")
    wid = sid * SC_CORES + cid
    lo = wid * RNG
    pltpu.sync_copy(h_hbm.at[pl.ds(lo, RNG)], acc.at[pl.ds(0, RNG)])
    pltpu.sync_copy(v_hbm, v_vm.at[pl.ds(0, NPAD)])
    pltpu.sync_copy(wa_hbm, wa_vm)
    dummy_s = jnp.full((16,), N, jnp.int32)
    dummy_d = jnp.full((16,), RNG, jnp.int32)

    def prefill(j, carry):
        ss[pl.ds(j * 16, 16)] = dummy_s
        sd[pl.ds(j * 16, 16)] = dummy_d
        return carry

    lax.fori_loop(0, (SCAP + 160) // 16, prefill, 0)

    tail = E - (NCH - 1) * CH

    def chunk_body(ch, pos):
        base = ch * CH
        pltpu.sync_copy(src_h.at[pl.ds(base, tail)], svm.at[pl.ds(0, tail)])
        pltpu.sync_copy(dst_h.at[pl.ds(base, tail)], dvm.at[pl.ds(0, tail)])

        @pl.when(ch < NCH - 1)
        def _():
            pltpu.sync_copy(src_h.at[pl.ds(base + tail, CH - tail)],
                            svm.at[pl.ds(tail, CH - tail)])
            pltpu.sync_copy(dst_h.at[pl.ds(base + tail, CH - tail)],
                            dvm.at[pl.ds(tail, CH - tail)])
        nsv = jnp.where(ch < NCH - 1, CH // 16, tail // 16)

        def sub_body(k, pos):
            s16 = svm[pl.ds(k * 16, 16)]
            d16 = dvm[pl.ds(k * 16, 16)]
            msk = (d16 >= lo) & (d16 < lo + RNG)
            cnt = jnp.sum(msk.astype(jnp.int32))
            p = jnp.minimum(pos, SCAP)
            plsc.store_compressed(ss.at[pl.ds(p, 16)], s16, mask=msk)
            plsc.store_compressed(sd.at[pl.ds(p, 16)], d16 - lo, mask=msk)
            return pos + cnt

        return lax.fori_loop(0, nsv, sub_body, pos)

    pos = lax.fori_loop(0, NCH, chunk_body, jnp.int32(0))
    pos = jnp.minimum(pos, SCAP)

    def flush(f, carry):
        pltpu.async_copy(h_hbm.at[ss.at[pl.ds(f * 128, 128)]], rows, sem).wait()

        def upd(j, carry):
            dl = _sload(sd, f * 128 + j)
            for cc in range(8):
                sl = pl.ds(cc * 16, 16)
                acc[dl, sl] = jnp.maximum(acc[dl, sl], rows[j, sl])
            return carry

        return lax.fori_loop(0, 128, upd, carry)

    lax.fori_loop(0, (pos + 127) // 128, flush, 0)

    def udot(r, carry):
        t = jnp.zeros((16,), jnp.float32)
        for cc in range(8):
            sl = pl.ds(cc * 16, 16)
            t = t + acc[r, sl] * wa_vm[sl]
        uu = jnp.sum(t)
        _sstore(u_vm, r, uu)
        _sstore(m_vm, r, _leaky(uu + _sload(v_vm, lo + r)))
        return carry

    lax.fori_loop(0, RNG, udot, 0)

    def mupd(j, carry):
        dl = _sload(sd, j)
        sc = _leaky(_sload(u_vm, dl) + _sload(v_vm, _sload(ss, j)))
        _sstore(m_vm, dl, jnp.maximum(_sload(m_vm, dl), sc))
        return carry

    lax.fori_loop(0, pos, mupd, 0)
    pltpu.sync_copy(u_vm.at[pl.ds(0, RNG)], u_out.at[pl.ds(lo, RNG)])
    pltpu.sync_copy(m_vm.at[pl.ds(0, RNG)], m_out.at[pl.ds(lo, RNG)])


def _poolmax_sc(hp, src_e, dst_e, v, wa):
    mesh = plsc.VectorSubcoreMesh(core_axis_name="c", subcore_axis_name="s",
                                  num_cores=SC_CORES, num_subcores=SC_TILES)
    return pl.kernel(
        _poolmax_body,
        out_type=(jax.ShapeDtypeStruct((NPAD,), jnp.float32),
                  jax.ShapeDtypeStruct((NPAD,), jnp.float32)),
        mesh=mesh,
        compiler_params=pltpu.CompilerParams(needs_layout_passes=False),
        scratch_types=[
            pltpu.VMEM((RNG + 8, D), jnp.float32),
            pltpu.VMEM((SCAP + 160,), jnp.int32),
            pltpu.VMEM((SCAP + 160,), jnp.int32),
            pltpu.VMEM((CH,), jnp.int32),
            pltpu.VMEM((CH,), jnp.int32),
            pltpu.VMEM((NPAD + 16,), jnp.float32),
            pltpu.VMEM((D,), jnp.float32),
            pltpu.VMEM((RNG + 16,), jnp.float32),
            pltpu.VMEM((RNG + 16,), jnp.float32),
            pltpu.VMEM((128, D), jnp.float32),
            pltpu.SemaphoreType.DMA,
        ],
    )(hp, src_e, dst_e, v, wa)


def _scalsum_body(src_h, dst_h, u_hbm, v_hbm, m_hbm, den_out, deg_out, e_out,
                  uvm, vvm, mvm, dacc, gacc, svm, dvm, ebuf):
    cid = lax.axis_index("c")
    sid = lax.axis_index("s")
    wid = sid * SC_CORES + cid
    pltpu.sync_copy(u_hbm, uvm)
    pltpu.sync_copy(v_hbm, vvm)
    pltpu.sync_copy(m_hbm, mvm)
    zz = jnp.zeros((16,), jnp.float32)

    def zinit(j, carry):
        dacc[pl.ds(j * 16, 16)] = zz
        gacc[pl.ds(j * 16, 16)] = zz
        return carry

    lax.fori_loop(0, NPAD // 16, zinit, 0)
    epw = E // NWORK
    ones = jnp.ones((16,), jnp.float32)

    def chunk_body(ch, carry):
        base = wid * epw + ch * CH
        csz = jnp.minimum(epw - ch * CH, CH)
        pltpu.sync_copy(src_h.at[pl.ds(base, 1808)], svm.at[pl.ds(0, 1808)])
        pltpu.sync_copy(dst_h.at[pl.ds(base, 1808)], dvm.at[pl.ds(0, 1808)])

        @pl.when(csz == CH)
        def _():
            pltpu.sync_copy(src_h.at[pl.ds(base + 1808, CH - 1808)],
                            svm.at[pl.ds(1808, CH - 1808)])
            pltpu.sync_copy(dst_h.at[pl.ds(base + 1808, CH - 1808)],
                            dvm.at[pl.ds(1808, CH - 1808)])

        def sub_body(k, carry):
            s16 = svm[pl.ds(k * 16, 16)]
            d16 = dvm[pl.ds(k * 16, 16)]
            ud = plsc.load_gather(uvm, [d16])
            vs = plsc.load_gather(vvm, [s16])
            md = plsc.load_gather(mvm, [d16])
            e = jnp.exp(_leaky(ud + vs) - md)
            ebuf[pl.ds(k * 16, 16)] = e
            plsc.addupdate_scatter(dacc, [d16], e)
            plsc.addupdate_scatter(gacc, [d16], ones)
            return carry

        lax.fori_loop(0, csz // 16, sub_body, carry)
        pltpu.sync_copy(ebuf.at[pl.ds(0, 1808)], e_out.at[pl.ds(base, 1808)])

        @pl.when(csz == CH)
        def _():
            pltpu.sync_copy(ebuf.at[pl.ds(1808, CH - 1808)],
                            e_out.at[pl.ds(base + 1808, CH - 1808)])
        return carry

    lax.fori_loop(0, (epw + CH - 1) // CH, chunk_body, 0)
    pltpu.sync_copy(dacc, den_out.at[wid])
    pltpu.sync_copy(gacc, deg_out.at[wid])


def _scalsum_sc(src_e, dst_e, u, v, m):
    mesh = plsc.VectorSubcoreMesh(core_axis_name="c", subcore_axis_name="s",
                                  num_cores=SC_CORES, num_subcores=SC_TILES)
    return pl.kernel(
        _scalsum_body,
        out_type=(jax.ShapeDtypeStruct((NWORK, NPAD), jnp.float32),
                  jax.ShapeDtypeStruct((NWORK, NPAD), jnp.float32),
                  jax.ShapeDtypeStruct((E,), jnp.float32)),
        mesh=mesh,
        compiler_params=pltpu.CompilerParams(needs_layout_passes=False),
        scratch_types=[
            pltpu.VMEM((NPAD,), jnp.float32),
            pltpu.VMEM((NPAD,), jnp.float32),
            pltpu.VMEM((NPAD,), jnp.float32),
            pltpu.VMEM((NPAD,), jnp.float32),
            pltpu.VMEM((NPAD,), jnp.float32),
            pltpu.VMEM((CH,), jnp.int32),
            pltpu.VMEM((CH,), jnp.int32),
            pltpu.VMEM((CH,), jnp.float32),
        ],
    )(src_e, dst_e, u, v, m)


def _wsegsum_body(h_hbm, srcp, dstp, e_hbm, r_hbm, zin, out,
                  acc, rvm, sidx, didx, ebuf, abuf, rows, sem):
    cid = lax.axis_index("c")
    sid = lax.axis_index("s")
    wid = sid * SC_CORES + cid
    r0 = sid * ROWS_PT
    pltpu.sync_copy(zin.at[pl.ds(r0, ROWS_PT)], acc.at[pl.ds(r0, ROWS_PT)])
    pltpu.sync_copy(r_hbm, rvm)
    plsc.subcore_barrier()
    wbase = wid * EPW

    def body(u, carry):
        base = wbase + u * 128
        pltpu.sync_copy(srcp.at[pl.ds(base, 128)], sidx)
        pltpu.sync_copy(dstp.at[pl.ds(base, 128)], didx)
        pltpu.sync_copy(e_hbm.at[pl.ds(base, 128)], ebuf)
        for k in range(8):
            d16 = didx[pl.ds(k * 16, 16)]
            rd = plsc.load_gather(rvm, [d16])
            abuf[pl.ds(k * 16, 16)] = ebuf[pl.ds(k * 16, 16)] * rd
        pltpu.async_copy(h_hbm.at[sidx], rows, sem).wait()

        def scale(j, carry):
            aj = _sload(abuf, j)
            for cc in range(8):
                sl = pl.ds(cc * 16, 16)
                rows[j, sl] = rows[j, sl] * aj
            return carry

        lax.fori_loop(0, 128, scale, 0)
        pltpu.sync_copy(rows, acc.at[didx], add=True)
        return carry

    lax.fori_loop(0, UNITS, body, 0)
    plsc.subcore_barrier()
    pltpu.sync_copy(acc.at[pl.ds(r0, ROWS_PT)], out.at[cid, pl.ds(r0, ROWS_PT)])


def _wsegsum_sc(hp, srcp, dstp, eattn, rden):
    zin = jnp.zeros((NPAD, D), jnp.float32)
    mesh = plsc.VectorSubcoreMesh(core_axis_name="c", subcore_axis_name="s",
                                  num_cores=SC_CORES, num_subcores=SC_TILES)
    return pl.kernel(
        _wsegsum_body,
        out_type=jax.ShapeDtypeStruct((SC_CORES, NPAD, D), jnp.float32),
        mesh=mesh,
        compiler_params=pltpu.CompilerParams(needs_layout_passes=False),
        scratch_types=[
            pltpu.VMEM_SHARED((NPAD, D), jnp.float32),
            pltpu.VMEM((NPAD,), jnp.float32),
            pltpu.VMEM((128,), jnp.int32),
            pltpu.VMEM((128,), jnp.int32),
            pltpu.VMEM((128,), jnp.float32),
            pltpu.VMEM((144,), jnp.float32),
            pltpu.VMEM((128, D), jnp.float32),
            pltpu.SemaphoreType.DMA,
        ],
    )(hp, srcp, dstp, eattn, rden, zin)


def _topk_body(c0_ref, aa_ref, b3_ref, deg_ref, sp_ref, kept_ref, w_ref):
    fit = (c0_ref[...] + aa_ref[...] - b3_ref[...]
           + deg_ref[...] * aa_ref[...] - jnp.sum(sp_ref[...], axis=0))
    bu = lax.bitcast_convert_type(fit, jnp.uint32)
    ku = jnp.where(fit >= 0.0, bu | jnp.uint32(0x80000000), ~bu)
    ri = lax.broadcasted_iota(jnp.int32, (80, 128), 0)
    ci = lax.broadcasted_iota(jnp.int32, (80, 128), 1)
    valid = (ri * 128 + ci) < N
    ku = jnp.where(valid, ku, jnp.uint32(0))

    def bit_step(i, t):
        t2 = t | (jnp.uint32(1) << (jnp.uint32(31) - i.astype(jnp.uint32)))
        cnt = jnp.sum((ku >= t2).astype(jnp.int32))
        return jnp.where(cnt >= KSEL, t2, t)

    t = lax.fori_loop(0, 32, bit_step, jnp.uint32(0))
    above = (ku > t).astype(jnp.float32)
    g_cnt = jnp.sum(above).astype(jnp.int32)
    tie = ((ku == t) & valid).astype(jnp.float32)
    ut128 = (lax.broadcasted_iota(jnp.int32, (128, 128), 0)
             <= lax.broadcasted_iota(jnp.int32, (128, 128), 1)).astype(jnp.float32)
    lt80 = (lax.broadcasted_iota(jnp.int32, (80, 80), 1)
            < lax.broadcasted_iota(jnp.int32, (80, 80), 0)).astype(jnp.float32)
    r1 = jnp.dot(tie, ut128, preferred_element_type=jnp.float32)
    rowsum = jnp.sum(tie, axis=1, keepdims=True)
    rp = jnp.dot(lt80, rowsum, preferred_element_type=jnp.float32)
    rank_excl = rp + r1 - tie
    need = (KSEL - g_cnt).astype(jnp.float32)
    kept = above + tie * (rank_excl < need).astype(jnp.float32)
    kept_ref[...] = kept
    w_ref[...] = kept / (1.0 + jnp.exp(-fit))


def _topk(c0, aa, b3, deg, s_parts):
    full = pl.BlockSpec((80, 128), lambda: (0, 0))
    return pl.pallas_call(
        _topk_body,
        in_specs=[full, full, full, full,
                  pl.BlockSpec((NWORK, 80, 128), lambda: (0, 0, 0))],
        out_specs=[full, full],
        out_shape=[jax.ShapeDtypeStruct((80, 128), jnp.float32),
                   jax.ShapeDtypeStruct((80, 128), jnp.float32)],
    )(c0, aa, b3, deg, s_parts)


def _readout_body(g_ref, kept_ref, batch_ref, clsw_ref, clsb_ref, acc_ref, out_ref):
    i = pl.program_id(0)

    @pl.when(i == 0)
    def _():
        acc_ref[...] = jnp.zeros_like(acc_ref)

    oh = (batch_ref[...] == lax.broadcasted_iota(jnp.int32, (1, 128), 1)
          ).astype(jnp.float32)
    gk = g_ref[...] * kept_ref[...]
    rhs = jnp.concatenate([gk, kept_ref[...], jnp.zeros((BLK, 127), jnp.float32)],
                          axis=1)
    acc_ref[...] += lax.dot_general(oh, rhs, (((0,), (0,)), ((), ())),
                                    preferred_element_type=jnp.float32)

    @pl.when(i == pl.num_programs(0) - 1)
    def _():
        sums = acc_ref[:, :D]
        cnt = acc_ref[:, D:D + 1]
        readout = sums / jnp.maximum(cnt, 1.0)
        out_ref[...] = jnp.dot(readout, clsw_ref[...],
                               preferred_element_type=jnp.float32) + clsb_ref[...]


def _readout(g, kept_col, batch_col, clsw_pad, clsb_pad):
    row_spec = pl.BlockSpec((BLK, D), lambda i: (i, 0))
    col_spec = pl.BlockSpec((BLK, 1), lambda i: (i, 0))
    return pl.pallas_call(
        _readout_body,
        grid=(NPAD // BLK,),
        in_specs=[row_spec, col_spec, col_spec,
                  pl.BlockSpec((D, 128), lambda i: (0, 0)),
                  pl.BlockSpec((1, 128), lambda i: (0, 0))],
        out_specs=[pl.BlockSpec((NG, 256), lambda i: (0, 0)),
                   pl.BlockSpec((NG, 128), lambda i: (0, 0))],
        out_shape=[jax.ShapeDtypeStruct((NG, 256), jnp.float32),
                   jax.ShapeDtypeStruct((NG, 128), jnp.float32)],
    )(g, kept_col, batch_col, clsw_pad, clsb_pad)


def _gin_mlp_body(scale, h_ref, a0_ref, a1_ref, w1_ref, b1_ref, w2_ref, b2_ref, o_ref):
    z = h_ref[...] + scale * (a0_ref[...] + a1_ref[...])
    z = jnp.maximum(jnp.dot(z, w1_ref[...], preferred_element_type=jnp.float32) + b1_ref[...], 0.0)
    o_ref[...] = jnp.maximum(jnp.dot(z, w2_ref[...], preferred_element_type=jnp.float32) + b2_ref[...], 0.0)


def _matcols_body(h_ref, w_ref, b_ref, o_ref):
    o_ref[...] = jnp.dot(h_ref[...], w_ref[...],
                         preferred_element_type=jnp.float32) + b_ref[...]


def _matcols(h, wcols, brow):
    row_spec = pl.BlockSpec((BLK, D), lambda i: (i, 0))
    return pl.pallas_call(
        _matcols_body,
        grid=(NPAD // BLK,),
        in_specs=[row_spec, pl.BlockSpec((D, 128), lambda i: (0, 0)),
                  pl.BlockSpec((1, 128), lambda i: (0, 0))],
        out_specs=pl.BlockSpec((BLK, 128), lambda i: (i, 0)),
        out_shape=jax.ShapeDtypeStruct((NPAD, 128), jnp.float32),
    )(h, wcols, brow)


def _poolred_body(dp_ref, gp_ref, u_ref, v_ref, m_ref, rden_ref, ws_ref, deg_ref):
    es = jnp.exp(_leaky(u_ref[...] + v_ref[...]) - m_ref[...])
    den = jnp.sum(dp_ref[...], axis=0, keepdims=True) + es
    rden = 1.0 / (den + 1e-16)
    rden_ref[...] = rden
    ws_ref[...] = es * rden
    deg_ref[...] = jnp.sum(gp_ref[...], axis=0, keepdims=True)


def _poolred(den_parts, deg_parts, u, v, m):
    cspec = pl.BlockSpec((1, 128), lambda i: (0, i))
    pspec = pl.BlockSpec((NWORK, 128), lambda i: (0, i))
    out1d = jax.ShapeDtypeStruct((1, NPAD), jnp.float32)
    return pl.pallas_call(
        _poolred_body,
        grid=(NPAD // 128,),
        in_specs=[pspec, pspec, cspec, cspec, cspec],
        out_specs=[cspec, cspec, cspec],
        out_shape=[out1d, out1d, out1d],
    )(den_parts, deg_parts, u.reshape(1, NPAD), v.reshape(1, NPAD),
      m.reshape(1, NPAD))


def _xcabc_body(p0_ref, p1_ref, h_ref, ws_ref, w_ref, b_ref, xc_ref, abc_ref):
    xc = p0_ref[...] + p1_ref[...] + h_ref[...] * ws_ref[...]
    xc_ref[...] = xc
    abc_ref[...] = jnp.dot(xc, w_ref[...],
                           preferred_element_type=jnp.float32) + b_ref[...]


def _xcabc(p0, p1, hp, ws_col, wcols, brow):
    row_spec = pl.BlockSpec((BLK, D), lambda i: (i, 0))
    return pl.pallas_call(
        _xcabc_body,
        grid=(NPAD // BLK,),
        in_specs=[row_spec, row_spec, row_spec,
                  pl.BlockSpec((BLK, 1), lambda i: (i, 0)),
                  pl.BlockSpec((D, 128), lambda i: (0, 0)),
                  pl.BlockSpec((1, 128), lambda i: (0, 0))],
        out_specs=[row_spec, pl.BlockSpec((BLK, 128), lambda i: (i, 0))],
        out_shape=[jax.ShapeDtypeStruct((NPAD, D), jnp.float32),
                   jax.ShapeDtypeStruct((NPAD, 128), jnp.float32)],
    )(p0, p1, hp, ws_col, wcols, brow)


def _gin_mlp(h, a0, a1, w1, b1, w2, b2, scale=1.0):
    grid = (NPAD // BLK,)
    row_spec = pl.BlockSpec((BLK, D), lambda i: (i, 0))
    w_spec = pl.BlockSpec((D, D), lambda i: (0, 0))
    b_spec = pl.BlockSpec((1, D), lambda i: (0, 0))
    return pl.pallas_call(
        functools.partial(_gin_mlp_body, scale),
        grid=grid,
        in_specs=[row_spec, row_spec, row_spec, w_spec, b_spec, w_spec, b_spec],
        out_specs=row_spec,
        out_shape=jax.ShapeDtypeStruct((NPAD, D), jnp.float32),
    )(h, a0, a1, w1, b1.reshape(1, D), w2, b2.reshape(1, D))


def _pad_rows(x):
    return jnp.pad(x, ((0, NPAD - N), (0, 0)))


def kernel(x, enc_W1, enc_b1, enc_W2, enc_b2, enc_W3, enc_b3, enc_W4, enc_b4,
           pool_lin_W, pool_lin_b, pool_att_W, pool_att_b,
           score_W1, score_b1, score_W2, score_W3,
           gnn_W1, gnn_b1, gnn_W2, gnn_b2, gnn_W3, gnn_b3, gnn_W4, gnn_b4,
           cls_W, cls_b, edge_index, batch):
    src, dst = edge_index[0], edge_index[1]
    epad = jnp.full((EPAD - src.shape[0],), N, jnp.int32)
    srcp = jnp.concatenate([src, epad])
    dstp = jnp.concatenate([dst, epad])

    def gin(hp, dp, w1, b1, w2, b2, scale=1.0):
        p = _segsum_sc(hp, srcp, dp)
        return _gin_mlp(hp, p[0], p[1], w1, b1, w2, b2, scale)

    hp = gin(_pad_rows(x), dstp, enc_W1, enc_b1, enc_W2, enc_b2)
    hp = gin(hp, dstp, enc_W3, enc_b3, enc_W4, enc_b4)
    h = hp[:N]

    a1 = pool_att_W[:D, 0]
    a2 = pool_att_W[D:, 0]
    wa = pool_lin_W @ a1
    vconst = pool_lin_b @ a1 + pool_att_b[0]
    wv = jnp.zeros((D, 128), jnp.float32).at[:, 0].set(a2)
    bv = jnp.zeros((1, 128), jnp.float32).at[0, 0].set(vconst)
    v = _matcols(hp, wv, bv)[:, 0]
    u, m = _poolmax_sc(hp, src, dst, v, wa)
    den_parts, deg_parts, e_arr = _scalsum_sc(src, dst, u, v, m)
    rden2, ws2, deg2 = _poolred(den_parts, deg_parts, u, v, m)
    eattn = jnp.concatenate([e_arr, jnp.zeros((EPAD - E,), jnp.float32)])
    p = _wsegsum_sc(hp, srcp, dstp, eattn, rden2.reshape(NPAD))
    wsc = (jnp.zeros((D, 128), jnp.float32)
           .at[:, 0].set(score_W1[:, 0])
           .at[:, 1].set(score_W2[:, 0])
           .at[:, 2].set(score_W3[:, 0]))
    bsc = jnp.zeros((1, 128), jnp.float32).at[0, 0].set(score_b1[0])
    xcp, abc = _xcabc(p[0], p[1], hp, ws2.reshape(NPAD, 1), wsc, bsc)
    b3 = abc[:, 2]
    s_jnp = jnp.pad(jax.ops.segment_sum(b3[:N][src], dst, num_segments=N),
                    (0, NPAD - N))
    s_parts = jnp.zeros((NWORK, NPAD), jnp.float32).at[0].set(s_jnp)
    kept2, w2 = _topk(abc[:, 0].reshape(80, 128), abc[:, 1].reshape(80, 128),
                      b3.reshape(80, 128), deg2.reshape(80, 128),
                      s_parts.reshape(NWORK, 80, 128))
    px = xcp * w2.reshape(NPAD, 1)
    kept = kept2.reshape(NPAD)[:N]
    em = kept[src] * kept[dst]
    c = float(1.0 / (1.0 + math.exp(-1.0)))
    dstm = jnp.concatenate([jnp.where(em > 0.5, dst, N).astype(jnp.int32), epad])

    gp = gin(px, dstm, gnn_W1, gnn_b1, gnn_W2, gnn_b2, scale=c)
    gp = gin(gp, dstm, gnn_W3, gnn_b3, gnn_W4, gnn_b4, scale=c)

    clsw_pad = jnp.zeros((D, 128), jnp.float32).at[:, :NC].set(cls_W)
    clsb_pad = jnp.zeros((1, 128), jnp.float32).at[0, :NC].set(cls_b)
    batch_col = jnp.pad(batch, (0, NPAD - N)).reshape(NPAD, 1)
    logits = _readout(gp, kept2.reshape(NPAD, 1), batch_col, clsw_pad, clsb_pad)[1]
    return logits[:, :NC]
